# baseline (device time: 34404 ns/iter reference)
import jax
import jax.numpy as jnp
from jax import lax
from jax.experimental import pallas as pl
from jax.experimental.pallas import tpu as pltpu

N_DEV = 4
N_TOK = 1024
D_IN = 512
D_OUT = 1024
N_EXP = 16
E_LOC = N_EXP // N_DEV
CAP = 51
BLK = N_TOK // N_DEV


def kernel(x, router_W, route_idx, expert_W):
    my_i = lax.axis_index("i")

    e = route_idx[:, 0]
    onehot = e[:, None] == jnp.arange(N_EXP, dtype=e.dtype)[None, :]
    pos = jnp.cumsum(onehot.astype(jnp.int32), axis=0)
    keep = onehot & (pos <= CAP)
    mask = lax.dynamic_slice(
        keep.astype(jnp.bfloat16), (0, my_i * E_LOC), (N_TOK, E_LOC)
    )

    def body(x_ref, w_ref, mask_ref, out_ref, partial_ref, recv_ref,
             send_sems, recv_sems):
        my = lax.axis_index("i")

        barrier = pltpu.get_barrier_semaphore()
        for k in range(1, N_DEV):
            pl.semaphore_signal(
                barrier, inc=1,
                device_id=((my + k) % N_DEV,),
                device_id_type=pl.DeviceIdType.MESH,
            )
        pl.semaphore_wait(barrier, N_DEV - 1)

        xb = x_ref[...].astype(jnp.bfloat16)
        m = mask_ref[...]
        acc = jnp.dot(
            xb * m[:, 0:1], w_ref[0].astype(jnp.bfloat16),
            preferred_element_type=jnp.float32,
        )
        for le in range(1, E_LOC):
            acc = acc + jnp.dot(
                xb * m[:, le:le + 1], w_ref[le].astype(jnp.bfloat16),
                preferred_element_type=jnp.float32,
            )
        partial_ref[...] = acc.astype(jnp.bfloat16)

        sends = []
        for k in range(1, N_DEV):
            dst = (my + k) % N_DEV
            rdma = pltpu.make_async_remote_copy(
                src_ref=partial_ref.at[pl.ds(dst * BLK, BLK), :],
                dst_ref=recv_ref.at[my],
                send_sem=send_sems.at[k],
                recv_sem=recv_sems.at[my],
                device_id=(dst,),
                device_id_type=pl.DeviceIdType.MESH,
            )
            rdma.start()
            sends.append(rdma)

        total = partial_ref[pl.ds(my * BLK, BLK), :].astype(jnp.float32)

        for k in range(1, N_DEV):
            src = (my + k) % N_DEV
            recv = pltpu.make_async_remote_copy(
                src_ref=partial_ref.at[pl.ds(0, BLK), :],
                dst_ref=recv_ref.at[src],
                send_sem=send_sems.at[0],
                recv_sem=recv_sems.at[src],
                device_id=(src,),
                device_id_type=pl.DeviceIdType.MESH,
            )
            recv.wait_recv()
            total = total + recv_ref[src].astype(jnp.float32)

        for s in sends:
            s.wait_send()

        out_ref[...] = total

    return pl.pallas_call(
        body,
        out_shape=jax.ShapeDtypeStruct((BLK, D_OUT), jnp.float32),
        in_specs=[
            pl.BlockSpec(memory_space=pltpu.VMEM),
            pl.BlockSpec(memory_space=pltpu.VMEM),
            pl.BlockSpec(memory_space=pltpu.VMEM),
        ],
        out_specs=pl.BlockSpec(memory_space=pltpu.VMEM),
        scratch_shapes=[
            pltpu.VMEM((N_TOK, D_OUT), jnp.bfloat16),
            pltpu.VMEM((N_DEV, BLK, D_OUT), jnp.bfloat16),
            pltpu.SemaphoreType.DMA((N_DEV,)),
            pltpu.SemaphoreType.DMA((N_DEV,)),
        ],
        compiler_params=pltpu.CompilerParams(collective_id=0),
    )(x, expert_W, mask)


# device time: 26862 ns/iter; 1.2808x vs baseline; 1.2808x over previous
import jax
import jax.numpy as jnp
from jax import lax
from jax.experimental import pallas as pl
from jax.experimental.pallas import tpu as pltpu

N_DEV = 4
N_TOK = 1024
D_IN = 512
D_OUT = 1024
N_EXP = 16
E_LOC = N_EXP // N_DEV
CAP = 51
BLK = N_TOK // N_DEV

_K_ORDER = (2, 1, 3)


def kernel(x, router_W, route_idx, expert_W):
    del router_W

    def body(x_ref, e_ref, w_ref, out_ref, mask_ref, sendbuf_ref, recv_ref,
             send_sems, recv_sems):
        my = lax.axis_index("i")

        barrier = pltpu.get_barrier_semaphore()
        for k in range(1, N_DEV):
            pl.semaphore_signal(
                barrier, inc=1,
                device_id=((my + k) % N_DEV,),
                device_id_type=pl.DeviceIdType.MESH,
            )
        pl.semaphore_wait(barrier, N_DEV - 1)

        e = e_ref[...]
        local_ids = my * E_LOC + lax.broadcasted_iota(jnp.int32, (1, E_LOC), 1)
        onehot = (e == local_ids).astype(jnp.float32)
        row = lax.broadcasted_iota(jnp.int32, (N_TOK, N_TOK), 0)
        col = lax.broadcasted_iota(jnp.int32, (N_TOK, N_TOK), 1)
        tril = (col <= row).astype(jnp.float32)
        pos = jnp.dot(tril, onehot, preferred_element_type=jnp.float32)
        mask_ref[...] = (onehot * (pos <= CAP)).astype(jnp.bfloat16)

        wb = [w_ref[le].astype(jnp.bfloat16) for le in range(E_LOC)]

        def compute_block(dst):
            r0 = dst * BLK
            xb = x_ref[pl.ds(r0, BLK), :].astype(jnp.bfloat16)
            mb = mask_ref[pl.ds(r0, BLK), :]
            acc = jnp.dot(xb * mb[:, 0:1], wb[0],
                          preferred_element_type=jnp.float32)
            for le in range(1, E_LOC):
                acc = acc + jnp.dot(xb * mb[:, le:le + 1], wb[le],
                                    preferred_element_type=jnp.float32)
            return acc

        sends = []
        for k in _K_ORDER:
            dst = (my + k) % N_DEV
            slot = k - 1
            sendbuf_ref[slot] = compute_block(dst).astype(jnp.bfloat16)
            rdma = pltpu.make_async_remote_copy(
                src_ref=sendbuf_ref.at[slot],
                dst_ref=recv_ref.at[my],
                send_sem=send_sems.at[slot],
                recv_sem=recv_sems.at[my],
                device_id=(dst,),
                device_id_type=pl.DeviceIdType.MESH,
            )
            rdma.start()
            sends.append(rdma)

        total = compute_block(my)

        for k in range(1, N_DEV):
            src = (my + k) % N_DEV
            recv = pltpu.make_async_remote_copy(
                src_ref=sendbuf_ref.at[0],
                dst_ref=recv_ref.at[src],
                send_sem=send_sems.at[0],
                recv_sem=recv_sems.at[src],
                device_id=(src,),
                device_id_type=pl.DeviceIdType.MESH,
            )
            recv.wait_recv()
            total = total + recv_ref[src].astype(jnp.float32)

        for s in sends:
            s.wait_send()

        out_ref[...] = total

    return pl.pallas_call(
        body,
        out_shape=jax.ShapeDtypeStruct((BLK, D_OUT), jnp.float32),
        in_specs=[
            pl.BlockSpec(memory_space=pltpu.VMEM),
            pl.BlockSpec(memory_space=pltpu.VMEM),
            pl.BlockSpec(memory_space=pltpu.VMEM),
        ],
        out_specs=pl.BlockSpec(memory_space=pltpu.VMEM),
        scratch_shapes=[
            pltpu.VMEM((N_TOK, E_LOC), jnp.bfloat16),
            pltpu.VMEM((N_DEV - 1, BLK, D_OUT), jnp.bfloat16),
            pltpu.VMEM((N_DEV, BLK, D_OUT), jnp.bfloat16),
            pltpu.SemaphoreType.DMA((N_DEV - 1,)),
            pltpu.SemaphoreType.DMA((N_DEV,)),
        ],
        compiler_params=pltpu.CompilerParams(collective_id=0),
    )(x, route_idx, expert_W)
